# Initial kernel scaffold; baseline (speedup 1.0000x reference)
import jax
import jax.numpy as jnp
from jax.experimental import pallas as pl


def kernel(rois, gt_boxes, gt_classes):
    raise NotImplementedError("write your pallas kernel here")


# speedup vs baseline: 2.7501x; 2.7501x over previous
import jax
import jax.numpy as jnp
from jax import lax
from jax.experimental import pallas as pl
from jax.experimental.pallas import tpu as pltpu

_B = 8
_N_GT = 100
_N = 20100
_ROWS = 160
_LANES = 128
_NP = _ROWS * _LANES
_GTL = 104
_K_POS = 64
_K_NEG = 192
_NS = 256


def _tc_body(rois_ref, gt_ref, noise_ref, planes_ref, slots_ref,
             ps_ref, ns_ref):
    f32 = jnp.float32
    g = gt_ref[0]
    gy0 = g[0]; gx0 = g[1]; gy1 = g[2]; gx1 = g[3]; gcls = g[4]
    ga = (gy1 - gy0) * (gx1 - gx0)
    gm2 = (jnp.maximum(jnp.maximum(gy0, gx0), jnp.maximum(gy1, gx1))
           < 0.0).astype(jnp.float32)

    def G(x):
        return x[None, None, :]

    iotag = lax.broadcasted_iota(jnp.int32, (1, 1, _GTL), 2)

    def chunk(c, _):
        s = pl.multiple_of(c * 8, 8)
        ry0 = rois_ref[0, 0, pl.ds(s, 8), :]
        rx0 = rois_ref[0, 1, pl.ds(s, 8), :]
        ry1 = rois_ref[0, 2, pl.ds(s, 8), :]
        rx1 = rois_ref[0, 3, pl.ds(s, 8), :]
        pn = noise_ref[0, 0, pl.ds(s, 8), :]
        nn = noise_ref[0, 1, pl.ds(s, 8), :]

        def A(x):
            return x[:, :, None]

        ih = jnp.maximum(jnp.minimum(A(ry1), G(gy1)) - jnp.maximum(A(ry0), G(gy0)), 0.0)
        iw = jnp.maximum(jnp.minimum(A(rx1), G(gx1)) - jnp.maximum(A(rx0), G(gx0)), 0.0)
        inter = ih * iw
        ra = (ry1 - ry0) * (rx1 - rx0)
        union = A(ra) + G(ga) - inter
        iou = jnp.where(union > 0.0, inter / jnp.maximum(union, 1e-8), 0.0)
        rm = (jnp.maximum(jnp.maximum(ry0, rx0), jnp.maximum(ry1, rx1))
              < 0.0).astype(jnp.float32)
        sim = jnp.where(A(rm) + G(gm2) > 0.0, -1.0, iou)
        best = jnp.max(sim, axis=-1)
        colcand = jnp.where(sim == best[:, :, None], iotag, jnp.int32(1 << 30))
        col = jnp.min(colcand, axis=-1)
        onehot = iotag == col[:, :, None]

        def pick(v):
            return jnp.sum(jnp.where(onehot, G(v), 0.0), axis=-1)

        my0 = pick(gy0); mx0 = pick(gx0); my1 = pick(gy1); mx1 = pick(gx1)
        mcls = pick(gcls)

        positive = best >= 0.5
        negative = jnp.logical_and(best >= 0.0, best < 0.5)

        ah = ry1 - ry0
        aw = rx1 - rx0
        acy = ry0 + 0.5 * ah
        acx = rx0 + 0.5 * aw
        bh = my1 - my0
        bw = mx1 - mx0
        bcy = my0 + 0.5 * bh
        bcx = mx0 + 0.5 * bw
        dy = ((bcy - acy) / jnp.maximum(ah, 1e-8)) / 0.1
        dx = ((bcx - acx) / jnp.maximum(aw, 1e-8)) / 0.1
        dh = jnp.log(jnp.maximum(bh, 1e-8) / jnp.maximum(ah, 1e-8)) / 0.2
        dw = jnp.log(jnp.maximum(bw, 1e-8) / jnp.maximum(aw, 1e-8)) / 0.2

        z = jnp.zeros_like(dy)
        planes_ref[0, 0, pl.ds(s, 8), :] = ry0
        planes_ref[0, 1, pl.ds(s, 8), :] = rx0
        planes_ref[0, 2, pl.ds(s, 8), :] = ry1
        planes_ref[0, 3, pl.ds(s, 8), :] = rx1
        planes_ref[0, 4, pl.ds(s, 8), :] = jnp.where(positive, dy, z)
        planes_ref[0, 5, pl.ds(s, 8), :] = jnp.where(positive, dx, z)
        planes_ref[0, 6, pl.ds(s, 8), :] = jnp.where(positive, dh, z)
        planes_ref[0, 7, pl.ds(s, 8), :] = jnp.where(positive, dw, z)
        planes_ref[0, 8, pl.ds(s, 8), :] = jnp.where(positive, mcls, z)
        planes_ref[0, 9, pl.ds(s, 8), :] = positive.astype(f32)
        ps_ref[pl.ds(s, 8), :] = jnp.where(positive, pn, -1.0)
        ns_ref[pl.ds(s, 8), :] = jnp.where(negative, nn, -1.0)
        return 0

    lax.fori_loop(0, _ROWS // 8, chunk, 0, unroll=2)
    PS = ps_ref[:, :]
    NSc = ns_ref[:, :]

    ip = (lax.broadcasted_iota(jnp.int32, (_ROWS, _LANES), 0) * _LANES
          + lax.broadcasted_iota(jnp.int32, (_ROWS, _LANES), 1))

    def topk_select(bits, k):
        T = jnp.int32(0)
        for bit in range(30, -1, -1):
            T2 = jnp.bitwise_or(T, jnp.int32(1 << bit))
            cnt = jnp.sum((bits >= T2).astype(jnp.int32))
            T = jnp.where(cnt >= k, T2, T)
        cgt = jnp.sum((bits > T).astype(jnp.int32))
        m = k - cgt
        E = bits == T
        r = jnp.int32(0)
        for bit in range(14, -1, -1):
            r2 = jnp.bitwise_or(r, jnp.int32(1 << bit))
            cr = jnp.sum(jnp.logical_and(E, ip < r2).astype(jnp.int32))
            r = jnp.where(cr < m, r2, r)
        tie = jnp.logical_and(jnp.logical_and(E, ip <= r), m > 0)
        return jnp.logical_or(bits > T, tie)

    pb = lax.bitcast_convert_type(PS, jnp.int32)
    nb = lax.bitcast_convert_type(NSc, jnp.int32)
    psel = topk_select(pb, _K_POS)
    nsel = topk_select(nb, _K_NEG)
    ind = jnp.logical_or(psel, nsel)
    indf = ind.astype(f32)

    U = (lax.broadcasted_iota(jnp.int32, (_LANES, _LANES), 0)
         <= lax.broadcasted_iota(jnp.int32, (_LANES, _LANES), 1)).astype(f32)
    rowcum = jnp.dot(indf, U, preferred_element_type=f32)
    rowtot = rowcum[:, _LANES - 1:_LANES]
    Lm = (lax.broadcasted_iota(jnp.int32, (_ROWS, _ROWS), 0)
          > lax.broadcasted_iota(jnp.int32, (_ROWS, _ROWS), 1)).astype(f32)
    carry = jnp.dot(Lm, rowtot, preferred_element_type=f32)
    rank_excl = carry + rowcum - indf
    selcnt = jnp.sum(indf)
    ipf = ip.astype(f32)
    slot = jnp.where(ind, rank_excl, selcnt + ipf - rank_excl)

    planes_ref[0, 10] = indf
    slots_ref[0, 0] = slot.astype(jnp.int32)


def _run_tc(rois4, gt5, noise, interpret=False):
    return pl.pallas_call(
        _tc_body,
        grid=(_B,),
        in_specs=[
            pl.BlockSpec((1, 4, _ROWS, _LANES), lambda b: (b, 0, 0, 0)),
            pl.BlockSpec((1, 8, _GTL), lambda b: (b, 0, 0)),
            pl.BlockSpec((1, 2, _ROWS, _LANES), lambda b: (b, 0, 0, 0)),
        ],
        out_specs=[
            pl.BlockSpec((1, 11, _ROWS, _LANES), lambda b: (b, 0, 0, 0)),
            pl.BlockSpec((1, 1, _ROWS, _LANES), lambda b: (b, 0, 0, 0)),
        ],
        out_shape=[
            jax.ShapeDtypeStruct((_B, 11, _ROWS, _LANES), jnp.float32),
            jax.ShapeDtypeStruct((_B, 1, _ROWS, _LANES), jnp.int32),
        ],
        scratch_shapes=[
            pltpu.VMEM((_ROWS, _LANES), jnp.float32),
            pltpu.VMEM((_ROWS, _LANES), jnp.float32),
        ],
        compiler_params=pltpu.CompilerParams(
            dimension_semantics=("arbitrary",)),
        interpret=interpret,
    )(rois4, gt5, noise)


def kernel(rois, gt_boxes, gt_classes):
    f32 = jnp.float32
    rois_all = jnp.concatenate([rois, gt_boxes], axis=1)
    pad = jnp.full((_B, _NP - _N, 4), -1.0, f32)
    rois_p = jnp.concatenate([rois_all, pad], axis=1)
    rois4 = rois_p.transpose(0, 2, 1).reshape(_B, 4, _ROWS, _LANES)

    gt5 = jnp.concatenate(
        [gt_boxes.transpose(0, 2, 1), gt_classes.transpose(0, 2, 1)], axis=1)
    gt5 = jnp.pad(gt5, ((0, 0), (0, 3), (0, _GTL - _N_GT)),
                  constant_values=-1.0)

    kp, kn = jax.random.split(jax.random.key(42))
    pn = jax.random.uniform(kp, (_B, _N), minval=1e-3, maxval=0.999)
    nn = jax.random.uniform(kn, (_B, _N), minval=1e-3, maxval=0.999)
    noise = jnp.stack([pn, nn], axis=1)
    noise = jnp.pad(noise, ((0, 0), (0, 0), (0, _NP - _N)))
    noise = noise.reshape(_B, 2, _ROWS, _LANES)

    planes, slots = _run_tc(rois4, gt5, noise)

    slots_f = slots.reshape(_B, _NP)
    sidx = jnp.argsort(slots_f, axis=1)[:, :_NS]
    pf = planes.reshape(_B, 11, _NP)
    gth = jnp.take_along_axis(pf, sidx[:, None, :], axis=2)
    sampled_rois = gth[:, 0:4].transpose(0, 2, 1)
    sampled_gt_boxes = gth[:, 4:8].transpose(0, 2, 1)
    sampled_box_weights = gth[:, 9:10].transpose(0, 2, 1)
    sampled_gt_classes = gth[:, 8:9].transpose(0, 2, 1)
    sampled_class_weights = gth[:, 10:11].transpose(0, 2, 1)
    return (sampled_rois, sampled_gt_boxes, sampled_box_weights,
            sampled_gt_classes, sampled_class_weights)


# speedup vs baseline: 6.0778x; 2.2100x over previous
import jax
import jax.numpy as jnp
from jax import lax
from jax.experimental import pallas as pl
from jax.experimental.pallas import tpu as pltpu

_B = 8
_N_GT = 100
_N = 20100
_ROWS = 160
_LANES = 128
_NP = _ROWS * _LANES
_GTL = 104
_K_POS = 64
_K_NEG = 192
_NS = 256


def _tc_body(rois_ref, gt_ref, noise_ref, planes_ref, slots_ref,
             ps_ref, ns_ref):
    f32 = jnp.float32

    def spl(k):
        gcol = gt_ref[0, k]
        return jnp.broadcast_to(gcol[:, :, None], (_GTL, 8, _LANES))

    gy0 = spl(0); gx0 = spl(1); gy1 = spl(2); gx1 = spl(3); gcls = spl(4)
    ga = (gy1 - gy0) * (gx1 - gx0)
    gm2 = (jnp.maximum(jnp.maximum(gy0, gx0), jnp.maximum(gy1, gx1))
           < 0.0).astype(f32)

    iotag = jnp.broadcast_to(
        lax.broadcasted_iota(jnp.int32, (_GTL, 1, 1), 0), (_GTL, 8, _LANES))

    def chunk(c, _):
        s = pl.multiple_of(c * 8, 8)
        ry0 = rois_ref[0, 0, pl.ds(s, 8), :]
        rx0 = rois_ref[0, 1, pl.ds(s, 8), :]
        ry1 = rois_ref[0, 2, pl.ds(s, 8), :]
        rx1 = rois_ref[0, 3, pl.ds(s, 8), :]
        pn = noise_ref[0, 0, pl.ds(s, 8), :]
        nn = noise_ref[0, 1, pl.ds(s, 8), :]

        def A(x):
            return x[None, :, :]

        ih = jnp.maximum(jnp.minimum(A(ry1), gy1) - jnp.maximum(A(ry0), gy0), 0.0)
        iw = jnp.maximum(jnp.minimum(A(rx1), gx1) - jnp.maximum(A(rx0), gx0), 0.0)
        inter = ih * iw
        ra = (ry1 - ry0) * (rx1 - rx0)
        union = A(ra) + ga - inter
        iou = jnp.where(union > 0.0, inter / jnp.maximum(union, 1e-8), 0.0)
        rm = (jnp.maximum(jnp.maximum(ry0, rx0), jnp.maximum(ry1, rx1))
              < 0.0).astype(f32)
        sim = jnp.where(A(rm) + gm2 > 0.0, -1.0, iou)
        best = jnp.max(sim, axis=0)
        colcand = jnp.where(sim == A(best), iotag, jnp.int32(1 << 30))
        col = jnp.min(colcand, axis=0)
        onehot = iotag == A(col)

        def pick(v):
            return jnp.sum(jnp.where(onehot, v, 0.0), axis=0)

        my0 = pick(gy0); mx0 = pick(gx0); my1 = pick(gy1); mx1 = pick(gx1)
        mcls = pick(gcls)

        positive = best >= 0.5
        negative = jnp.logical_and(best >= 0.0, best < 0.5)

        ah = ry1 - ry0
        aw = rx1 - rx0
        acy = ry0 + 0.5 * ah
        acx = rx0 + 0.5 * aw
        bh = my1 - my0
        bw = mx1 - mx0
        bcy = my0 + 0.5 * bh
        bcx = mx0 + 0.5 * bw
        dy = ((bcy - acy) / jnp.maximum(ah, 1e-8)) / 0.1
        dx = ((bcx - acx) / jnp.maximum(aw, 1e-8)) / 0.1
        dh = jnp.log(jnp.maximum(bh, 1e-8) / jnp.maximum(ah, 1e-8)) / 0.2
        dw = jnp.log(jnp.maximum(bw, 1e-8) / jnp.maximum(aw, 1e-8)) / 0.2

        z = jnp.zeros_like(dy)
        planes_ref[0, 0, pl.ds(s, 8), :] = ry0
        planes_ref[0, 1, pl.ds(s, 8), :] = rx0
        planes_ref[0, 2, pl.ds(s, 8), :] = ry1
        planes_ref[0, 3, pl.ds(s, 8), :] = rx1
        planes_ref[0, 4, pl.ds(s, 8), :] = jnp.where(positive, dy, z)
        planes_ref[0, 5, pl.ds(s, 8), :] = jnp.where(positive, dx, z)
        planes_ref[0, 6, pl.ds(s, 8), :] = jnp.where(positive, dh, z)
        planes_ref[0, 7, pl.ds(s, 8), :] = jnp.where(positive, dw, z)
        planes_ref[0, 8, pl.ds(s, 8), :] = jnp.where(positive, mcls, z)
        planes_ref[0, 9, pl.ds(s, 8), :] = positive.astype(f32)
        ps_ref[pl.ds(s, 8), :] = jnp.where(positive, pn, -1.0)
        ns_ref[pl.ds(s, 8), :] = jnp.where(negative, nn, -1.0)
        return 0

    lax.fori_loop(0, _ROWS // 8, chunk, 0, unroll=2)
    PS = ps_ref[:, :]
    NSc = ns_ref[:, :]

    ip = (lax.broadcasted_iota(jnp.int32, (_ROWS, _LANES), 0) * _LANES
          + lax.broadcasted_iota(jnp.int32, (_ROWS, _LANES), 1))

    def topk_select(bits, k):
        T = jnp.int32(0)
        for bit in range(30, -1, -1):
            T2 = jnp.bitwise_or(T, jnp.int32(1 << bit))
            cnt = jnp.sum((bits >= T2).astype(jnp.int32))
            T = jnp.where(cnt >= k, T2, T)
        cgt = jnp.sum((bits > T).astype(jnp.int32))
        m = k - cgt
        E = bits == T
        r = jnp.int32(0)
        for bit in range(14, -1, -1):
            r2 = jnp.bitwise_or(r, jnp.int32(1 << bit))
            cr = jnp.sum(jnp.logical_and(E, ip < r2).astype(jnp.int32))
            r = jnp.where(cr < m, r2, r)
        tie = jnp.logical_and(jnp.logical_and(E, ip <= r), m > 0)
        return jnp.logical_or(bits > T, tie)

    pb = lax.bitcast_convert_type(PS, jnp.int32)
    nb = lax.bitcast_convert_type(NSc, jnp.int32)
    psel = topk_select(pb, _K_POS)
    nsel = topk_select(nb, _K_NEG)
    ind = jnp.logical_or(psel, nsel)
    indf = ind.astype(f32)

    U = (lax.broadcasted_iota(jnp.int32, (_LANES, _LANES), 0)
         <= lax.broadcasted_iota(jnp.int32, (_LANES, _LANES), 1)).astype(f32)
    rowcum = jnp.dot(indf, U, preferred_element_type=f32)
    rowtot = rowcum[:, _LANES - 1:_LANES]
    Lm = (lax.broadcasted_iota(jnp.int32, (_ROWS, _ROWS), 0)
          > lax.broadcasted_iota(jnp.int32, (_ROWS, _ROWS), 1)).astype(f32)
    carry = jnp.dot(Lm, rowtot, preferred_element_type=f32)
    rank_excl = carry + rowcum - indf
    selcnt = jnp.sum(indf)
    ipf = ip.astype(f32)
    slot = jnp.where(ind, rank_excl, selcnt + ipf - rank_excl)

    planes_ref[0, 10] = indf
    slots_ref[0, 0] = slot.astype(jnp.int32)


def _run_tc(rois4, gt5, noise, interpret=False):
    return pl.pallas_call(
        _tc_body,
        grid=(_B,),
        in_specs=[
            pl.BlockSpec((1, 4, _ROWS, _LANES), lambda b: (b, 0, 0, 0)),
            pl.BlockSpec((1, 5, _GTL, 1), lambda b: (b, 0, 0, 0)),
            pl.BlockSpec((1, 2, _ROWS, _LANES), lambda b: (b, 0, 0, 0)),
        ],
        out_specs=[
            pl.BlockSpec((1, 11, _ROWS, _LANES), lambda b: (b, 0, 0, 0)),
            pl.BlockSpec((1, 1, _ROWS, _LANES), lambda b: (b, 0, 0, 0)),
        ],
        out_shape=[
            jax.ShapeDtypeStruct((_B, 11, _ROWS, _LANES), jnp.float32),
            jax.ShapeDtypeStruct((_B, 1, _ROWS, _LANES), jnp.int32),
        ],
        scratch_shapes=[
            pltpu.VMEM((_ROWS, _LANES), jnp.float32),
            pltpu.VMEM((_ROWS, _LANES), jnp.float32),
        ],
        compiler_params=pltpu.CompilerParams(
            dimension_semantics=("arbitrary",)),
        interpret=interpret,
    )(rois4, gt5, noise)


def kernel(rois, gt_boxes, gt_classes):
    f32 = jnp.float32
    rois_all = jnp.concatenate([rois, gt_boxes], axis=1)
    pad = jnp.full((_B, _NP - _N, 4), -1.0, f32)
    rois_p = jnp.concatenate([rois_all, pad], axis=1)
    rois4 = rois_p.transpose(0, 2, 1).reshape(_B, 4, _ROWS, _LANES)

    gt5 = jnp.concatenate(
        [gt_boxes.transpose(0, 2, 1), gt_classes.transpose(0, 2, 1)], axis=1)
    gt5 = jnp.pad(gt5, ((0, 0), (0, 0), (0, _GTL - _N_GT)),
                  constant_values=-1.0)[..., None]

    kp, kn = jax.random.split(jax.random.key(42))
    pn = jax.random.uniform(kp, (_B, _N), minval=1e-3, maxval=0.999)
    nn = jax.random.uniform(kn, (_B, _N), minval=1e-3, maxval=0.999)
    noise = jnp.stack([pn, nn], axis=1)
    noise = jnp.pad(noise, ((0, 0), (0, 0), (0, _NP - _N)))
    noise = noise.reshape(_B, 2, _ROWS, _LANES)

    planes, slots = _run_tc(rois4, gt5, noise)

    slots_f = slots.reshape(_B, _NP)
    sidx = jnp.argsort(slots_f, axis=1)[:, :_NS]
    pf = planes.reshape(_B, 11, _NP)
    gth = jnp.take_along_axis(pf, sidx[:, None, :], axis=2)
    sampled_rois = gth[:, 0:4].transpose(0, 2, 1)
    sampled_gt_boxes = gth[:, 4:8].transpose(0, 2, 1)
    sampled_box_weights = gth[:, 9:10].transpose(0, 2, 1)
    sampled_gt_classes = gth[:, 8:9].transpose(0, 2, 1)
    sampled_class_weights = gth[:, 10:11].transpose(0, 2, 1)
    return (sampled_rois, sampled_gt_boxes, sampled_box_weights,
            sampled_gt_classes, sampled_class_weights)


# speedup vs baseline: 9.2385x; 1.5200x over previous
import functools

import jax
import jax.numpy as jnp
from jax import lax
from jax.experimental import pallas as pl
from jax.experimental.pallas import tpu as pltpu
from jax.experimental.pallas import tpu_sc as plsc

_B = 8
_N_GT = 100
_N = 20100
_ROWS = 160
_LANES = 128
_NP = _ROWS * _LANES
_GTL = 104
_K_POS = 64
_K_NEG = 192
_NS = 256


def _tc_body(rois_ref, gt_ref, noise_ref, planes_ref, slots_ref,
             ps_ref, ns_ref):
    f32 = jnp.float32

    def spl(k):
        gcol = gt_ref[0, k]
        return jnp.broadcast_to(gcol[:, :, None], (_GTL, 8, _LANES))

    gy0 = spl(0); gx0 = spl(1); gy1 = spl(2); gx1 = spl(3); gcls = spl(4)
    ga = (gy1 - gy0) * (gx1 - gx0)
    gm2 = (jnp.maximum(jnp.maximum(gy0, gx0), jnp.maximum(gy1, gx1))
           < 0.0).astype(f32)

    iotag = jnp.broadcast_to(
        lax.broadcasted_iota(jnp.int32, (_GTL, 1, 1), 0), (_GTL, 8, _LANES))

    def chunk(c, _):
        s = pl.multiple_of(c * 8, 8)
        ry0 = rois_ref[0, 0, pl.ds(s, 8), :]
        rx0 = rois_ref[0, 1, pl.ds(s, 8), :]
        ry1 = rois_ref[0, 2, pl.ds(s, 8), :]
        rx1 = rois_ref[0, 3, pl.ds(s, 8), :]
        pn = noise_ref[0, 0, pl.ds(s, 8), :]
        nn = noise_ref[0, 1, pl.ds(s, 8), :]

        def A(x):
            return x[None, :, :]

        ih = jnp.maximum(jnp.minimum(A(ry1), gy1) - jnp.maximum(A(ry0), gy0), 0.0)
        iw = jnp.maximum(jnp.minimum(A(rx1), gx1) - jnp.maximum(A(rx0), gx0), 0.0)
        inter = ih * iw
        ra = (ry1 - ry0) * (rx1 - rx0)
        union = A(ra) + ga - inter
        iou = jnp.where(union > 0.0, inter / jnp.maximum(union, 1e-8), 0.0)
        rm = (jnp.maximum(jnp.maximum(ry0, rx0), jnp.maximum(ry1, rx1))
              < 0.0).astype(f32)
        sim = jnp.where(A(rm) + gm2 > 0.0, -1.0, iou)
        best = jnp.max(sim, axis=0)
        colcand = jnp.where(sim == A(best), iotag, jnp.int32(1 << 30))
        col = jnp.min(colcand, axis=0)
        onehot = iotag == A(col)

        def pick(v):
            return jnp.sum(jnp.where(onehot, v, 0.0), axis=0)

        my0 = pick(gy0); mx0 = pick(gx0); my1 = pick(gy1); mx1 = pick(gx1)
        mcls = pick(gcls)

        positive = best >= 0.5
        negative = jnp.logical_and(best >= 0.0, best < 0.5)

        ah = ry1 - ry0
        aw = rx1 - rx0
        acy = ry0 + 0.5 * ah
        acx = rx0 + 0.5 * aw
        bh = my1 - my0
        bw = mx1 - mx0
        bcy = my0 + 0.5 * bh
        bcx = mx0 + 0.5 * bw
        dy = ((bcy - acy) / jnp.maximum(ah, 1e-8)) / 0.1
        dx = ((bcx - acx) / jnp.maximum(aw, 1e-8)) / 0.1
        dh = jnp.log(jnp.maximum(bh, 1e-8) / jnp.maximum(ah, 1e-8)) / 0.2
        dw = jnp.log(jnp.maximum(bw, 1e-8) / jnp.maximum(aw, 1e-8)) / 0.2

        z = jnp.zeros_like(dy)
        planes_ref[0, 0, pl.ds(s, 8), :] = ry0
        planes_ref[0, 1, pl.ds(s, 8), :] = rx0
        planes_ref[0, 2, pl.ds(s, 8), :] = ry1
        planes_ref[0, 3, pl.ds(s, 8), :] = rx1
        planes_ref[0, 4, pl.ds(s, 8), :] = jnp.where(positive, dy, z)
        planes_ref[0, 5, pl.ds(s, 8), :] = jnp.where(positive, dx, z)
        planes_ref[0, 6, pl.ds(s, 8), :] = jnp.where(positive, dh, z)
        planes_ref[0, 7, pl.ds(s, 8), :] = jnp.where(positive, dw, z)
        planes_ref[0, 8, pl.ds(s, 8), :] = jnp.where(positive, mcls, z)
        planes_ref[0, 9, pl.ds(s, 8), :] = positive.astype(f32)
        ps_ref[pl.ds(s, 8), :] = jnp.where(positive, pn, -1.0)
        ns_ref[pl.ds(s, 8), :] = jnp.where(negative, nn, -1.0)
        return 0

    lax.fori_loop(0, _ROWS // 8, chunk, 0, unroll=2)
    PS = ps_ref[:, :]
    NSc = ns_ref[:, :]

    ip = (lax.broadcasted_iota(jnp.int32, (_ROWS, _LANES), 0) * _LANES
          + lax.broadcasted_iota(jnp.int32, (_ROWS, _LANES), 1))

    def topk_select(bits, k):
        T = jnp.int32(0)
        for bit in range(30, -1, -1):
            T2 = jnp.bitwise_or(T, jnp.int32(1 << bit))
            cnt = jnp.sum((bits >= T2).astype(jnp.int32))
            T = jnp.where(cnt >= k, T2, T)
        cgt = jnp.sum((bits > T).astype(jnp.int32))
        m = k - cgt
        E = bits == T
        r = jnp.int32(0)
        for bit in range(14, -1, -1):
            r2 = jnp.bitwise_or(r, jnp.int32(1 << bit))
            cr = jnp.sum(jnp.logical_and(E, ip < r2).astype(jnp.int32))
            r = jnp.where(cr < m, r2, r)
        tie = jnp.logical_and(jnp.logical_and(E, ip <= r), m > 0)
        return jnp.logical_or(bits > T, tie)

    pb = lax.bitcast_convert_type(PS, jnp.int32)
    nb = lax.bitcast_convert_type(NSc, jnp.int32)
    psel = topk_select(pb, _K_POS)
    nsel = topk_select(nb, _K_NEG)
    ind = jnp.logical_or(psel, nsel)
    indf = ind.astype(f32)

    U = (lax.broadcasted_iota(jnp.int32, (_LANES, _LANES), 0)
         <= lax.broadcasted_iota(jnp.int32, (_LANES, _LANES), 1)).astype(f32)
    rowcum = jnp.dot(indf, U, preferred_element_type=f32)
    rowtot = rowcum[:, _LANES - 1:_LANES]
    Lm = (lax.broadcasted_iota(jnp.int32, (_ROWS, _ROWS), 0)
          > lax.broadcasted_iota(jnp.int32, (_ROWS, _ROWS), 1)).astype(f32)
    carry = jnp.dot(Lm, rowtot, preferred_element_type=f32)
    rank_excl = carry + rowcum - indf
    selcnt = jnp.sum(indf)
    ipf = ip.astype(f32)
    slot = jnp.where(ind, rank_excl, selcnt + ipf - rank_excl)

    planes_ref[0, 10] = indf
    slots_ref[0, 0] = slot.astype(jnp.int32)


def _run_tc(rois4, gt5, noise, interpret=False):
    return pl.pallas_call(
        _tc_body,
        grid=(_B,),
        in_specs=[
            pl.BlockSpec((1, 4, _ROWS, _LANES), lambda b: (b, 0, 0, 0)),
            pl.BlockSpec((1, 5, _GTL, 1), lambda b: (b, 0, 0, 0)),
            pl.BlockSpec((1, 2, _ROWS, _LANES), lambda b: (b, 0, 0, 0)),
        ],
        out_specs=[
            pl.BlockSpec((1, 11, _ROWS, _LANES), lambda b: (b, 0, 0, 0)),
            pl.BlockSpec((1, 1, _ROWS, _LANES), lambda b: (b, 0, 0, 0)),
        ],
        out_shape=[
            jax.ShapeDtypeStruct((_B, 11, _ROWS, _LANES), jnp.float32),
            jax.ShapeDtypeStruct((_B, 1, _ROWS, _LANES), jnp.int32),
        ],
        scratch_shapes=[
            pltpu.VMEM((_ROWS, _LANES), jnp.float32),
            pltpu.VMEM((_ROWS, _LANES), jnp.float32),
        ],
        compiler_params=pltpu.CompilerParams(
            dimension_semantics=("arbitrary",)),
        interpret=interpret,
    )(rois4, gt5, noise)


_NPL = 11
_SHARD = _NP // 16
_TASKS = 4 * _NPL * 2


def _sc_tail(slots2, planes4):
    mesh = plsc.VectorSubcoreMesh(core_axis_name="c", subcore_axis_name="s")

    @functools.partial(
        pl.kernel, mesh=mesh,
        out_type=jax.ShapeDtypeStruct((88, 256), jnp.float32),
        scratch_types=[
            pltpu.VMEM((4, _SHARD), jnp.int32),
            pltpu.VMEM((1024,), jnp.int32),
            pltpu.VMEM((256,), jnp.int32),
            pltpu.VMEM((256,), jnp.int32),
            pltpu.VMEM((256,), jnp.int32),
            pltpu.VMEM((_NP,), jnp.float32),
            pltpu.VMEM((256,), jnp.float32),
            pltpu.VMEM_SHARED((16, 1024), jnp.int32),
            pltpu.VMEM_SHARED((4, 256), jnp.int32),
        ],
        compiler_params=pltpu.CompilerParams(needs_layout_passes=False),
    )
    def sck(slots_hbm, planes_hbm, out_hbm,
            slots_v, local_v, acc_v, tmp_v, idxf_v, plane_v, out_v,
            sh_all, sh_idx):
        c = lax.axis_index("c")
        s = lax.axis_index("s")
        i16 = lax.iota(jnp.int32, 16)
        z16 = jnp.zeros((16,), jnp.int32)

        def zloc(i, _):
            local_v[pl.ds(i * 16, 16)] = z16
            return 0
        lax.fori_loop(0, 64, zloc, 0)

        base = s * _SHARD
        for k in range(4):
            pltpu.sync_copy(slots_hbm.at[2 * k + c, pl.ds(base, _SHARD)],
                            slots_v.at[k])

        def scan(i, _):
            off = i * 16
            for k in range(4):
                sv = slots_v[k, pl.ds(off, 16)]
                iglob = i16 + (base + off)
                plsc.store_scatter(local_v, [sv + (k * 256)], iglob,
                                   mask=sv < 256)
            return 0
        lax.fori_loop(0, _SHARD // 16, scan, 0)

        pltpu.sync_copy(local_v, sh_all.at[s])
        plsc.subcore_barrier()

        @pl.when(s < 4)
        def _():
            def zacc(i, _):
                acc_v[pl.ds(i * 16, 16)] = z16
                return 0
            lax.fori_loop(0, 16, zacc, 0)
            for src in range(16):
                pltpu.sync_copy(sh_all.at[src, pl.ds(s * 256, 256)], tmp_v)

                def add(i, _):
                    acc_v[pl.ds(i * 16, 16)] = (
                        acc_v[pl.ds(i * 16, 16)] + tmp_v[pl.ds(i * 16, 16)])
                    return 0
                lax.fori_loop(0, 16, add, 0)
            pltpu.sync_copy(acc_v, sh_idx.at[s])
        plsc.subcore_barrier()

        for j in range(3):
            t = s + 16 * j

            @pl.when(t < 4 * _NPL)
            def _():
                k = t // _NPL
                p = t % _NPL
                tid = (2 * k + c) * _NPL + p
                pltpu.sync_copy(planes_hbm.at[tid], plane_v)
                pltpu.sync_copy(sh_idx.at[k], idxf_v)

                def gat(q, _):
                    ii = idxf_v[pl.ds(q * 16, 16)]
                    out_v[pl.ds(q * 16, 16)] = plsc.load_gather(
                        plane_v, [ii])
                    return 0
                lax.fori_loop(0, 16, gat, 0)
                pltpu.sync_copy(out_v, out_hbm.at[tid])

    return sck(slots2, planes4)


def kernel(rois, gt_boxes, gt_classes):
    f32 = jnp.float32
    rois_all = jnp.concatenate([rois, gt_boxes], axis=1)
    pad = jnp.full((_B, _NP - _N, 4), -1.0, f32)
    rois_p = jnp.concatenate([rois_all, pad], axis=1)
    rois4 = rois_p.transpose(0, 2, 1).reshape(_B, 4, _ROWS, _LANES)

    gt5 = jnp.concatenate(
        [gt_boxes.transpose(0, 2, 1), gt_classes.transpose(0, 2, 1)], axis=1)
    gt5 = jnp.pad(gt5, ((0, 0), (0, 0), (0, _GTL - _N_GT)),
                  constant_values=-1.0)[..., None]

    kp, kn = jax.random.split(jax.random.key(42))
    pn = jax.random.uniform(kp, (_B, _N), minval=1e-3, maxval=0.999)
    nn = jax.random.uniform(kn, (_B, _N), minval=1e-3, maxval=0.999)
    noise = jnp.stack([pn, nn], axis=1)
    noise = jnp.pad(noise, ((0, 0), (0, 0), (0, _NP - _N)))
    noise = noise.reshape(_B, 2, _ROWS, _LANES)

    planes, slots = _run_tc(rois4, gt5, noise)

    slots2 = slots.reshape(_B, _NP)
    planes4 = planes.reshape(_B * _NPL, _NP)
    gth = _sc_tail(slots2, planes4).reshape(_B, _NPL, _NS)
    sampled_rois = gth[:, 0:4].transpose(0, 2, 1)
    sampled_gt_boxes = gth[:, 4:8].transpose(0, 2, 1)
    sampled_box_weights = gth[:, 9:10].transpose(0, 2, 1)
    sampled_gt_classes = gth[:, 8:9].transpose(0, 2, 1)
    sampled_class_weights = gth[:, 10:11].transpose(0, 2, 1)
    return (sampled_rois, sampled_gt_boxes, sampled_box_weights,
            sampled_gt_classes, sampled_class_weights)
